# baseline (speedup 1.0000x reference)
import functools

import jax
import jax.numpy as jnp
from jax import lax
from jax.experimental import pallas as pl
from jax.experimental.pallas import tpu as pltpu
from jax.experimental.pallas import tpu_sc as plsc

N = 10000
D = 128
E = 320000
NC = 2
NS = 16
NW = NC * NS
EPT = E // NW
K = 50
NCH = EPT // K
G = 40
NGRP = NCH // G
RPT = 624
TAIL = N - NS * RPT
TAILOFF = NS * RPT

_mesh = plsc.VectorSubcoreMesh(core_axis_name="c", subcore_axis_name="s")


@functools.partial(
    pl.kernel,
    out_type=jax.ShapeDtypeStruct((NC, N, D), jnp.float32),
    mesh=_mesh,
    scratch_types=[
        pltpu.VMEM_SHARED((N, D), jnp.float32),
        pltpu.VMEM((G, K), jnp.int32),
        pltpu.VMEM((G, K), jnp.int32),
        pltpu.VMEM((K, D), jnp.float32),
        pltpu.VMEM((K, D), jnp.float32),
        pltpu.VMEM((K, D), jnp.float32),
        pltpu.VMEM((K, D), jnp.float32),
        pltpu.SemaphoreType.DMA,
        pltpu.SemaphoreType.DMA,
        pltpu.SemaphoreType.DMA,
        pltpu.SemaphoreType.DMA,
        pltpu.SemaphoreType.DMA,
        pltpu.SemaphoreType.DMA,
        pltpu.SemaphoreType.DMA,
        pltpu.SemaphoreType.DMA,
    ],
)
def _sc_propagate(g_hbm, src_hbm, dst_hbm, z_hbm, out_hbm,
                  acc, srcg, dstg, r0, r1, r2, r3,
                  gs0, gs1, gs2, gs3, ss0, ss1, ss2, ss3):
    c = lax.axis_index("c")
    s = lax.axis_index("s")
    wid = c * NS + s
    pltpu.sync_copy(z_hbm, acc.at[pl.ds(s * RPT, RPT)])

    @pl.when(s == 0)
    def _():
        pltpu.sync_copy(z_hbm.at[pl.ds(0, TAIL)], acc.at[pl.ds(TAILOFF, TAIL)])

    plsc.subcore_barrier()

    rows = (r0, r1, r2, r3)
    gsem = (gs0, gs1, gs2, gs3)
    ssem = (ss0, ss1, ss2, ss3)

    def g_start(i, b):
        pltpu.async_copy(g_hbm.at[srcg.at[i]], rows[b], gsem[b])

    def g_wait(i, b):
        pltpu.make_async_copy(g_hbm.at[srcg.at[i]], rows[b], gsem[b]).wait()

    def s_start(i, b):
        pltpu.async_copy(rows[b], acc.at[dstg.at[i]], ssem[b], add=True)

    def s_wait(i, b):
        pltpu.make_async_copy(rows[b], acc.at[dstg.at[i]], ssem[b]).wait()

    @pl.loop(0, NGRP)
    def _(grp):
        pltpu.sync_copy(src_hbm.at[wid].at[grp], srcg)
        pltpu.sync_copy(dst_hbm.at[wid].at[grp], dstg)
        g_start(0, 0)
        g_start(1, 1)
        g_wait(0, 0); s_start(0, 0); g_start(2, 2)
        g_wait(1, 1); s_start(1, 1); g_start(3, 3)

        @pl.loop(0, (G - 4) // 4)
        def _(t):
            base = 4 * t
            for u in range(4):
                i = base + 2 + u
                b = (2 + u) % 4
                g_wait(i, b)
                s_start(i, b)
                s_wait(i - 2, (b - 2) % 4)
                g_start(i + 2, (b + 2) % 4)

        g_wait(G - 2, (G - 2) % 4); s_start(G - 2, (G - 2) % 4)
        s_wait(G - 4, (G - 4) % 4)
        g_wait(G - 1, (G - 1) % 4); s_start(G - 1, (G - 1) % 4)
        s_wait(G - 3, (G - 3) % 4)
        s_wait(G - 2, (G - 2) % 4)
        s_wait(G - 1, (G - 1) % 4)

    plsc.subcore_barrier()
    pltpu.sync_copy(acc.at[pl.ds(s * RPT, RPT)],
                    out_hbm.at[c].at[pl.ds(s * RPT, RPT)])

    @pl.when(s == 0)
    def _():
        pltpu.sync_copy(acc.at[pl.ds(TAILOFF, TAIL)],
                        out_hbm.at[c].at[pl.ds(TAILOFF, TAIL)])


@functools.partial(
    pl.kernel,
    out_type=jax.ShapeDtypeStruct((NC, N, D), jnp.float32),
    mesh=_mesh,
    scratch_types=[
        pltpu.VMEM_SHARED((N, D), jnp.float32),
        pltpu.VMEM((G, K), jnp.int32),
        pltpu.VMEM((K, D), jnp.float32),
    ],
)
def _sc_degree(dst_hbm, ones_hbm, z_hbm, out_hbm, acc, dstg, ones_v):
    c = lax.axis_index("c")
    s = lax.axis_index("s")
    wid = c * NS + s
    pltpu.sync_copy(z_hbm, acc.at[pl.ds(s * RPT, RPT)])
    pltpu.sync_copy(ones_hbm, ones_v)

    @pl.when(s == 0)
    def _():
        pltpu.sync_copy(z_hbm.at[pl.ds(0, TAIL)], acc.at[pl.ds(TAILOFF, TAIL)])

    plsc.subcore_barrier()

    @pl.loop(0, NGRP)
    def _(grp):
        pltpu.sync_copy(dst_hbm.at[wid].at[grp], dstg)

        @pl.loop(0, G)
        def _(j):
            pltpu.sync_copy(ones_v, acc.at[dstg.at[j]], add=True)

    plsc.subcore_barrier()
    pltpu.sync_copy(acc.at[pl.ds(s * RPT, RPT)],
                    out_hbm.at[c].at[pl.ds(s * RPT, RPT)])

    @pl.when(s == 0)
    def _():
        pltpu.sync_copy(acc.at[pl.ds(TAILOFF, TAIL)],
                        out_hbm.at[c].at[pl.ds(TAILOFF, TAIL)])


def _deg_finish(d0, d1, x):
    BN = 1000

    def body(d0_ref, d1_ref, x_ref, dv_ref, g0_ref):
        deg = d0_ref[:, :1] + d1_ref[:, :1] + 1.0
        dv = lax.rsqrt(deg)
        dv_ref[...] = jnp.broadcast_to(dv, dv_ref.shape)
        g0_ref[...] = dv * x_ref[...]

    return pl.pallas_call(
        body,
        grid=(N // BN,),
        in_specs=[pl.BlockSpec((BN, D), lambda i: (i, 0)),
                  pl.BlockSpec((BN, D), lambda i: (i, 0)),
                  pl.BlockSpec((BN, D), lambda i: (i, 0))],
        out_specs=[pl.BlockSpec((BN, D), lambda i: (i, 0)),
                   pl.BlockSpec((BN, D), lambda i: (i, 0))],
        out_shape=[jax.ShapeDtypeStruct((N, D), jnp.float32)] * 2,
    )(d0, d1, x)


def _combine(p0, p1, g, dinv):
    C = g.shape[1]
    BN = 1000

    def body(p0_ref, p1_ref, g_ref, dv_ref, h_ref, g2_ref):
        dv = dv_ref[:, :1]
        h = dv * (p0_ref[...] + p1_ref[...] + g_ref[...])
        h_ref[...] = h
        g2_ref[...] = dv * h

    return pl.pallas_call(
        body,
        grid=(N // BN,),
        in_specs=[pl.BlockSpec((BN, C), lambda i: (i, 0)),
                  pl.BlockSpec((BN, C), lambda i: (i, 0)),
                  pl.BlockSpec((BN, C), lambda i: (i, 0)),
                  pl.BlockSpec((BN, D), lambda i: (i, 0))],
        out_specs=[pl.BlockSpec((BN, C), lambda i: (i, 0)),
                   pl.BlockSpec((BN, C), lambda i: (i, 0))],
        out_shape=[jax.ShapeDtypeStruct((N, C), jnp.float32)] * 2,
    )(p0, p1, g, dinv)


def _mm(x, w, b_out=None, *, dinv=None, b_in=None, in_relu=False,
        out_relu=False):
    din, dout = w.shape
    BN = 1000
    n_in = 2 + (b_out is not None) + (dinv is not None) + (b_in is not None)
    n_out = 1 + (dinv is not None)

    def body(*refs):
        i = 0
        x_ref = refs[i]; i += 1
        w_ref = refs[i]; i += 1
        bo_ref = None
        bi_ref = None
        dv_ref = None
        if b_out is not None:
            bo_ref = refs[i]; i += 1
        if b_in is not None:
            bi_ref = refs[i]; i += 1
        if dinv is not None:
            dv_ref = refs[i]; i += 1
        y_ref = refs[i]; i += 1
        xb = x_ref[...]
        if b_in is not None:
            xb = xb + bi_ref[...]
        if in_relu:
            xb = jnp.maximum(xb, 0.0)
        y = jnp.dot(xb, w_ref[...], preferred_element_type=jnp.float32)
        if b_out is not None:
            y = y + bo_ref[...]
        if out_relu:
            y = jnp.maximum(y, 0.0)
        y_ref[...] = y
        if dinv is not None:
            refs[i][...] = dv_ref[...] * y

    in_specs = [pl.BlockSpec((BN, din), lambda i: (i, 0)),
                pl.BlockSpec((din, dout), lambda i: (0, 0))]
    args = [x, w]
    if b_out is not None:
        in_specs.append(pl.BlockSpec((1, dout), lambda i: (0, 0)))
        args.append(b_out.reshape(1, dout))
    if b_in is not None:
        in_specs.append(pl.BlockSpec((1, din), lambda i: (0, 0)))
        args.append(b_in.reshape(1, din))
    if dinv is not None:
        in_specs.append(pl.BlockSpec((BN, D), lambda i: (i, 0)))
        args.append(dinv)

    out = pl.pallas_call(
        body,
        grid=(N // BN,),
        in_specs=in_specs,
        out_specs=[pl.BlockSpec((BN, dout), lambda i: (i, 0))] * n_out,
        out_shape=[jax.ShapeDtypeStruct((N, dout), jnp.float32)] * n_out,
    )(*args)
    return out if n_out == 2 else (out[0], None)


def kernel(x, edge_index, W0_1, b0_1, W0_2, b0_2, W0_3, b0_3,
           W1_1, b1_1, W1_2, b1_2, W1_3, b1_3, W_head, b_head):
    src = edge_index[0].reshape(NW, NGRP, G, K)
    dst = edge_index[1].reshape(NW, NGRP, G, K)
    z128 = jnp.zeros((RPT, D), jnp.float32)
    ones128 = jnp.ones((K, D), jnp.float32)

    dg = _sc_degree(dst, ones128, z128)
    dinv, g0 = _deg_finish(dg[0], dg[1], x)

    def prop(g):
        p = _sc_propagate(g, src, dst, z128)
        return p[0], p[1]

    pa, pb = prop(g0)
    h1, g1 = _combine(pa, pb, g0, dinv)
    pa, pb = prop(g1)
    h2, g2 = _combine(pa, pb, g1, dinv)
    pa, pb = prop(g2)
    h3, _ = _combine(pa, pb, g2, dinv)

    _, gh1 = _mm(h1, W0_1, b0_1, dinv=dinv, out_relu=True)
    _, gh2 = _mm(h2, W0_2, b0_2, dinv=dinv, out_relu=True)
    _, gh3 = _mm(h3, W0_3, b0_3, dinv=dinv, out_relu=True)

    t1_blocks = []
    for gh in (gh1, gh2, gh3):
        pa, pb = prop(gh)
        tb, _ = _combine(pa, pb, gh, dinv)
        t1_blocks.append(tb)
    t1 = jnp.concatenate(t1_blocks, axis=1)

    out1, _ = _mm(t1, W1_1)
    _, gu1 = _mm(t1, W1_2, dinv=dinv)
    _, gu2 = _mm(t1, W1_3, dinv=dinv)

    pa, pb = prop(gu1)
    out2, _ = _combine(pa, pb, gu1, dinv)
    pa, pb = prop(gu2)
    _, gv = _combine(pa, pb, gu2, dinv)
    pa, pb = prop(gv)
    out3, _ = _combine(pa, pb, gv, dinv)

    h2pre = jnp.concatenate([out1, out2, out3], axis=1)
    bcat = jnp.concatenate([b1_1, b1_2, b1_3])
    y, _ = _mm(h2pre, W_head, b_head, b_in=bcat, in_relu=True)
    return y


# speedup vs baseline: 1.1355x; 1.1355x over previous
import functools

import jax
import jax.numpy as jnp
from jax import lax
from jax.experimental import pallas as pl
from jax.experimental.pallas import tpu as pltpu
from jax.experimental.pallas import tpu_sc as plsc

N = 10000
D = 128
E = 320000
NC = 2
NS = 16
NW = NC * NS
EPT = E // NW
K = 100
NCH = EPT // K
G = 25
NGRP = NCH // G
RPT = 624
TAIL = N - NS * RPT
TAILOFF = NS * RPT

_mesh = plsc.VectorSubcoreMesh(core_axis_name="c", subcore_axis_name="s")


@functools.partial(
    pl.kernel,
    out_type=jax.ShapeDtypeStruct((NC, N, D), jnp.float32),
    mesh=_mesh,
    scratch_types=[
        pltpu.VMEM_SHARED((N, D), jnp.float32),
        pltpu.VMEM((G, K), jnp.int32),
        pltpu.VMEM((G, K), jnp.int32),
        pltpu.VMEM((K, D), jnp.float32),
        pltpu.VMEM((K, D), jnp.float32),
        pltpu.SemaphoreType.DMA,
        pltpu.SemaphoreType.DMA,
    ],
)
def _sc_propagate(g_hbm, src_hbm, dst_hbm, z_hbm, out_hbm,
                  acc, srcg, dstg, rows0, rows1, sem0, sem1):
    c = lax.axis_index("c")
    s = lax.axis_index("s")
    wid = c * NS + s
    pltpu.sync_copy(z_hbm, acc.at[pl.ds(s * RPT, RPT)])

    @pl.when(s == 0)
    def _():
        pltpu.sync_copy(z_hbm.at[pl.ds(0, TAIL)], acc.at[pl.ds(TAILOFF, TAIL)])

    plsc.subcore_barrier()

    rows = (rows0, rows1)
    sems = (sem0, sem1)

    @pl.loop(0, NGRP)
    def _(grp):
        pltpu.sync_copy(src_hbm.at[wid].at[grp], srcg)
        pltpu.sync_copy(dst_hbm.at[wid].at[grp], dstg)
        pltpu.async_copy(g_hbm.at[srcg.at[0]], rows0, sem0)
        pltpu.async_copy(g_hbm.at[srcg.at[1]], rows1, sem1)

        @pl.loop(0, G - 3, step=2)
        def _(j):
            for b in range(2):
                pltpu.make_async_copy(g_hbm.at[srcg.at[j + b]],
                                      rows[b], sems[b]).wait()
                pltpu.sync_copy(rows[b], acc.at[dstg.at[j + b]], add=True)
                pltpu.async_copy(g_hbm.at[srcg.at[j + b + 2]], rows[b], sems[b])

        pltpu.make_async_copy(g_hbm.at[srcg.at[G - 3]], rows0, sem0).wait()
        pltpu.sync_copy(rows0, acc.at[dstg.at[G - 3]], add=True)
        pltpu.async_copy(g_hbm.at[srcg.at[G - 1]], rows0, sem0)
        pltpu.make_async_copy(g_hbm.at[srcg.at[G - 2]], rows1, sem1).wait()
        pltpu.sync_copy(rows1, acc.at[dstg.at[G - 2]], add=True)
        pltpu.make_async_copy(g_hbm.at[srcg.at[G - 1]], rows0, sem0).wait()
        pltpu.sync_copy(rows0, acc.at[dstg.at[G - 1]], add=True)

    plsc.subcore_barrier()
    pltpu.sync_copy(acc.at[pl.ds(s * RPT, RPT)],
                    out_hbm.at[c].at[pl.ds(s * RPT, RPT)])

    @pl.when(s == 0)
    def _():
        pltpu.sync_copy(acc.at[pl.ds(TAILOFF, TAIL)],
                        out_hbm.at[c].at[pl.ds(TAILOFF, TAIL)])


@functools.partial(
    pl.kernel,
    out_type=jax.ShapeDtypeStruct((NC, N, D), jnp.float32),
    mesh=_mesh,
    scratch_types=[
        pltpu.VMEM_SHARED((N, D), jnp.float32),
        pltpu.VMEM((G, K), jnp.int32),
        pltpu.VMEM((K, D), jnp.float32),
    ],
)
def _sc_degree(dst_hbm, ones_hbm, z_hbm, out_hbm, acc, dstg, ones_v):
    c = lax.axis_index("c")
    s = lax.axis_index("s")
    wid = c * NS + s
    pltpu.sync_copy(z_hbm, acc.at[pl.ds(s * RPT, RPT)])
    pltpu.sync_copy(ones_hbm, ones_v)

    @pl.when(s == 0)
    def _():
        pltpu.sync_copy(z_hbm.at[pl.ds(0, TAIL)], acc.at[pl.ds(TAILOFF, TAIL)])

    plsc.subcore_barrier()

    @pl.loop(0, NGRP)
    def _(grp):
        pltpu.sync_copy(dst_hbm.at[wid].at[grp], dstg)

        @pl.loop(0, G)
        def _(j):
            pltpu.sync_copy(ones_v, acc.at[dstg.at[j]], add=True)

    plsc.subcore_barrier()
    pltpu.sync_copy(acc.at[pl.ds(s * RPT, RPT)],
                    out_hbm.at[c].at[pl.ds(s * RPT, RPT)])

    @pl.when(s == 0)
    def _():
        pltpu.sync_copy(acc.at[pl.ds(TAILOFF, TAIL)],
                        out_hbm.at[c].at[pl.ds(TAILOFF, TAIL)])


def _deg_finish(d0, d1, x):
    BN = 1000

    def body(d0_ref, d1_ref, x_ref, dv_ref, g0_ref):
        deg = d0_ref[:, :1] + d1_ref[:, :1] + 1.0
        dv = lax.rsqrt(deg)
        dv_ref[...] = jnp.broadcast_to(dv, dv_ref.shape)
        g0_ref[...] = dv * x_ref[...]

    return pl.pallas_call(
        body,
        grid=(N // BN,),
        in_specs=[pl.BlockSpec((BN, D), lambda i: (i, 0)),
                  pl.BlockSpec((BN, D), lambda i: (i, 0)),
                  pl.BlockSpec((BN, D), lambda i: (i, 0))],
        out_specs=[pl.BlockSpec((BN, D), lambda i: (i, 0)),
                   pl.BlockSpec((BN, D), lambda i: (i, 0))],
        out_shape=[jax.ShapeDtypeStruct((N, D), jnp.float32)] * 2,
    )(d0, d1, x)


def _combine(p0, p1, g, dinv):
    C = g.shape[1]
    BN = 1000

    def body(p0_ref, p1_ref, g_ref, dv_ref, h_ref, g2_ref):
        dv = dv_ref[:, :1]
        h = dv * (p0_ref[...] + p1_ref[...] + g_ref[...])
        h_ref[...] = h
        g2_ref[...] = dv * h

    return pl.pallas_call(
        body,
        grid=(N // BN,),
        in_specs=[pl.BlockSpec((BN, C), lambda i: (i, 0)),
                  pl.BlockSpec((BN, C), lambda i: (i, 0)),
                  pl.BlockSpec((BN, C), lambda i: (i, 0)),
                  pl.BlockSpec((BN, D), lambda i: (i, 0))],
        out_specs=[pl.BlockSpec((BN, C), lambda i: (i, 0)),
                   pl.BlockSpec((BN, C), lambda i: (i, 0))],
        out_shape=[jax.ShapeDtypeStruct((N, C), jnp.float32)] * 2,
    )(p0, p1, g, dinv)


def _mm(x, w, b_out=None, *, dinv=None, b_in=None, in_relu=False,
        out_relu=False):
    din, dout = w.shape
    BN = 1000
    n_in = 2 + (b_out is not None) + (dinv is not None) + (b_in is not None)
    n_out = 1 + (dinv is not None)

    def body(*refs):
        i = 0
        x_ref = refs[i]; i += 1
        w_ref = refs[i]; i += 1
        bo_ref = None
        bi_ref = None
        dv_ref = None
        if b_out is not None:
            bo_ref = refs[i]; i += 1
        if b_in is not None:
            bi_ref = refs[i]; i += 1
        if dinv is not None:
            dv_ref = refs[i]; i += 1
        y_ref = refs[i]; i += 1
        xb = x_ref[...]
        if b_in is not None:
            xb = xb + bi_ref[...]
        if in_relu:
            xb = jnp.maximum(xb, 0.0)
        y = jnp.dot(xb, w_ref[...], preferred_element_type=jnp.float32)
        if b_out is not None:
            y = y + bo_ref[...]
        if out_relu:
            y = jnp.maximum(y, 0.0)
        y_ref[...] = y
        if dinv is not None:
            refs[i][...] = dv_ref[...] * y

    in_specs = [pl.BlockSpec((BN, din), lambda i: (i, 0)),
                pl.BlockSpec((din, dout), lambda i: (0, 0))]
    args = [x, w]
    if b_out is not None:
        in_specs.append(pl.BlockSpec((1, dout), lambda i: (0, 0)))
        args.append(b_out.reshape(1, dout))
    if b_in is not None:
        in_specs.append(pl.BlockSpec((1, din), lambda i: (0, 0)))
        args.append(b_in.reshape(1, din))
    if dinv is not None:
        in_specs.append(pl.BlockSpec((BN, D), lambda i: (i, 0)))
        args.append(dinv)

    out = pl.pallas_call(
        body,
        grid=(N // BN,),
        in_specs=in_specs,
        out_specs=[pl.BlockSpec((BN, dout), lambda i: (i, 0))] * n_out,
        out_shape=[jax.ShapeDtypeStruct((N, dout), jnp.float32)] * n_out,
    )(*args)
    return out if n_out == 2 else (out[0], None)


def kernel(x, edge_index, W0_1, b0_1, W0_2, b0_2, W0_3, b0_3,
           W1_1, b1_1, W1_2, b1_2, W1_3, b1_3, W_head, b_head):
    src = edge_index[0].reshape(NW, NGRP, G, K)
    dst = edge_index[1].reshape(NW, NGRP, G, K)
    z128 = jnp.zeros((RPT, D), jnp.float32)
    ones128 = jnp.ones((K, D), jnp.float32)

    dg = _sc_degree(dst, ones128, z128)
    dinv, g0 = _deg_finish(dg[0], dg[1], x)

    def prop(g):
        p = _sc_propagate(g, src, dst, z128)
        return p[0], p[1]

    pa, pb = prop(g0)
    h1, g1 = _combine(pa, pb, g0, dinv)
    pa, pb = prop(g1)
    h2, g2 = _combine(pa, pb, g1, dinv)
    pa, pb = prop(g2)
    h3, _ = _combine(pa, pb, g2, dinv)

    _, gh1 = _mm(h1, W0_1, b0_1, dinv=dinv, out_relu=True)
    _, gh2 = _mm(h2, W0_2, b0_2, dinv=dinv, out_relu=True)
    _, gh3 = _mm(h3, W0_3, b0_3, dinv=dinv, out_relu=True)

    t1_blocks = []
    for gh in (gh1, gh2, gh3):
        pa, pb = prop(gh)
        tb, _ = _combine(pa, pb, gh, dinv)
        t1_blocks.append(tb)
    t1 = jnp.concatenate(t1_blocks, axis=1)

    out1, _ = _mm(t1, W1_1)
    _, gu1 = _mm(t1, W1_2, dinv=dinv)
    _, gu2 = _mm(t1, W1_3, dinv=dinv)

    pa, pb = prop(gu1)
    out2, _ = _combine(pa, pb, gu1, dinv)
    pa, pb = prop(gu2)
    _, gv = _combine(pa, pb, gu2, dinv)
    pa, pb = prop(gv)
    out3, _ = _combine(pa, pb, gv, dinv)

    h2pre = jnp.concatenate([out1, out2, out3], axis=1)
    bcat = jnp.concatenate([b1_1, b1_2, b1_3])
    y, _ = _mm(h2pre, W_head, b_head, b_in=bcat, in_relu=True)
    return y


# speedup vs baseline: 1.1685x; 1.0290x over previous
import functools

import jax
import jax.numpy as jnp
from jax import lax
from jax.experimental import pallas as pl
from jax.experimental.pallas import tpu as pltpu
from jax.experimental.pallas import tpu_sc as plsc

N = 10000
D = 128
E = 320000
NC = 2
NS = 16
NW = NC * NS
EPT = E // NW
K = 125
NCH = EPT // K
G = 20
NGRP = NCH // G
RPT = 624
TAIL = N - NS * RPT
TAILOFF = NS * RPT

_mesh = plsc.VectorSubcoreMesh(core_axis_name="c", subcore_axis_name="s")


@functools.partial(
    pl.kernel,
    out_type=jax.ShapeDtypeStruct((NC, N, D), jnp.float32),
    mesh=_mesh,
    scratch_types=[
        pltpu.VMEM_SHARED((N, D), jnp.float32),
        pltpu.VMEM((G, K), jnp.int32),
        pltpu.VMEM((G, K), jnp.int32),
        pltpu.VMEM((K, D), jnp.float32),
        pltpu.VMEM((K, D), jnp.float32),
        pltpu.SemaphoreType.DMA,
        pltpu.SemaphoreType.DMA,
    ],
)
def _sc_propagate(g_hbm, src_hbm, dst_hbm, z_hbm, out_hbm,
                  acc, srcg, dstg, rows0, rows1, sem0, sem1):
    c = lax.axis_index("c")
    s = lax.axis_index("s")
    wid = c * NS + s
    pltpu.sync_copy(z_hbm, acc.at[pl.ds(s * RPT, RPT)])

    @pl.when(s == 0)
    def _():
        pltpu.sync_copy(z_hbm.at[pl.ds(0, TAIL)], acc.at[pl.ds(TAILOFF, TAIL)])

    plsc.subcore_barrier()

    rows = (rows0, rows1)
    sems = (sem0, sem1)

    @pl.loop(0, NGRP)
    def _(grp):
        pltpu.sync_copy(src_hbm.at[wid].at[grp], srcg)
        pltpu.sync_copy(dst_hbm.at[wid].at[grp], dstg)
        pltpu.async_copy(g_hbm.at[srcg.at[0]], rows0, sem0)
        pltpu.async_copy(g_hbm.at[srcg.at[1]], rows1, sem1)

        @pl.loop(0, G - 2, step=2)
        def _(j):
            for b in range(2):
                pltpu.make_async_copy(g_hbm.at[srcg.at[j + b]],
                                      rows[b], sems[b]).wait()
                pltpu.sync_copy(rows[b], acc.at[dstg.at[j + b]], add=True)
                pltpu.async_copy(g_hbm.at[srcg.at[j + b + 2]], rows[b], sems[b])

        for b in range(2):
            pltpu.make_async_copy(g_hbm.at[srcg.at[G - 2 + b]],
                                  rows[b], sems[b]).wait()
            pltpu.sync_copy(rows[b], acc.at[dstg.at[G - 2 + b]], add=True)

    plsc.subcore_barrier()
    pltpu.sync_copy(acc.at[pl.ds(s * RPT, RPT)],
                    out_hbm.at[c].at[pl.ds(s * RPT, RPT)])

    @pl.when(s == 0)
    def _():
        pltpu.sync_copy(acc.at[pl.ds(TAILOFF, TAIL)],
                        out_hbm.at[c].at[pl.ds(TAILOFF, TAIL)])


@functools.partial(
    pl.kernel,
    out_type=jax.ShapeDtypeStruct((NC, N, D), jnp.float32),
    mesh=_mesh,
    scratch_types=[
        pltpu.VMEM_SHARED((N, D), jnp.float32),
        pltpu.VMEM((G, K), jnp.int32),
        pltpu.VMEM((K, D), jnp.float32),
    ],
)
def _sc_degree(dst_hbm, ones_hbm, z_hbm, out_hbm, acc, dstg, ones_v):
    c = lax.axis_index("c")
    s = lax.axis_index("s")
    wid = c * NS + s
    pltpu.sync_copy(z_hbm, acc.at[pl.ds(s * RPT, RPT)])
    pltpu.sync_copy(ones_hbm, ones_v)

    @pl.when(s == 0)
    def _():
        pltpu.sync_copy(z_hbm.at[pl.ds(0, TAIL)], acc.at[pl.ds(TAILOFF, TAIL)])

    plsc.subcore_barrier()

    @pl.loop(0, NGRP)
    def _(grp):
        pltpu.sync_copy(dst_hbm.at[wid].at[grp], dstg)

        @pl.loop(0, G)
        def _(j):
            pltpu.sync_copy(ones_v, acc.at[dstg.at[j]], add=True)

    plsc.subcore_barrier()
    pltpu.sync_copy(acc.at[pl.ds(s * RPT, RPT)],
                    out_hbm.at[c].at[pl.ds(s * RPT, RPT)])

    @pl.when(s == 0)
    def _():
        pltpu.sync_copy(acc.at[pl.ds(TAILOFF, TAIL)],
                        out_hbm.at[c].at[pl.ds(TAILOFF, TAIL)])


def _deg_finish(d0, d1, x):
    BN = 1000

    def body(d0_ref, d1_ref, x_ref, dv_ref, g0_ref):
        deg = d0_ref[:, :1] + d1_ref[:, :1] + 1.0
        dv = lax.rsqrt(deg)
        dv_ref[...] = jnp.broadcast_to(dv, dv_ref.shape)
        g0_ref[...] = dv * x_ref[...]

    return pl.pallas_call(
        body,
        grid=(N // BN,),
        in_specs=[pl.BlockSpec((BN, D), lambda i: (i, 0)),
                  pl.BlockSpec((BN, D), lambda i: (i, 0)),
                  pl.BlockSpec((BN, D), lambda i: (i, 0))],
        out_specs=[pl.BlockSpec((BN, D), lambda i: (i, 0)),
                   pl.BlockSpec((BN, D), lambda i: (i, 0))],
        out_shape=[jax.ShapeDtypeStruct((N, D), jnp.float32)] * 2,
    )(d0, d1, x)


def _combine(p0, p1, g, dinv):
    C = g.shape[1]
    BN = 1000

    def body(p0_ref, p1_ref, g_ref, dv_ref, h_ref, g2_ref):
        dv = dv_ref[:, :1]
        h = dv * (p0_ref[...] + p1_ref[...] + g_ref[...])
        h_ref[...] = h
        g2_ref[...] = dv * h

    return pl.pallas_call(
        body,
        grid=(N // BN,),
        in_specs=[pl.BlockSpec((BN, C), lambda i: (i, 0)),
                  pl.BlockSpec((BN, C), lambda i: (i, 0)),
                  pl.BlockSpec((BN, C), lambda i: (i, 0)),
                  pl.BlockSpec((BN, D), lambda i: (i, 0))],
        out_specs=[pl.BlockSpec((BN, C), lambda i: (i, 0)),
                   pl.BlockSpec((BN, C), lambda i: (i, 0))],
        out_shape=[jax.ShapeDtypeStruct((N, C), jnp.float32)] * 2,
    )(p0, p1, g, dinv)


def _mm(x, w, b_out=None, *, dinv=None, b_in=None, in_relu=False,
        out_relu=False):
    din, dout = w.shape
    BN = 1000
    n_in = 2 + (b_out is not None) + (dinv is not None) + (b_in is not None)
    n_out = 1 + (dinv is not None)

    def body(*refs):
        i = 0
        x_ref = refs[i]; i += 1
        w_ref = refs[i]; i += 1
        bo_ref = None
        bi_ref = None
        dv_ref = None
        if b_out is not None:
            bo_ref = refs[i]; i += 1
        if b_in is not None:
            bi_ref = refs[i]; i += 1
        if dinv is not None:
            dv_ref = refs[i]; i += 1
        y_ref = refs[i]; i += 1
        xb = x_ref[...]
        if b_in is not None:
            xb = xb + bi_ref[...]
        if in_relu:
            xb = jnp.maximum(xb, 0.0)
        y = jnp.dot(xb, w_ref[...], preferred_element_type=jnp.float32)
        if b_out is not None:
            y = y + bo_ref[...]
        if out_relu:
            y = jnp.maximum(y, 0.0)
        y_ref[...] = y
        if dinv is not None:
            refs[i][...] = dv_ref[...] * y

    in_specs = [pl.BlockSpec((BN, din), lambda i: (i, 0)),
                pl.BlockSpec((din, dout), lambda i: (0, 0))]
    args = [x, w]
    if b_out is not None:
        in_specs.append(pl.BlockSpec((1, dout), lambda i: (0, 0)))
        args.append(b_out.reshape(1, dout))
    if b_in is not None:
        in_specs.append(pl.BlockSpec((1, din), lambda i: (0, 0)))
        args.append(b_in.reshape(1, din))
    if dinv is not None:
        in_specs.append(pl.BlockSpec((BN, D), lambda i: (i, 0)))
        args.append(dinv)

    out = pl.pallas_call(
        body,
        grid=(N // BN,),
        in_specs=in_specs,
        out_specs=[pl.BlockSpec((BN, dout), lambda i: (i, 0))] * n_out,
        out_shape=[jax.ShapeDtypeStruct((N, dout), jnp.float32)] * n_out,
    )(*args)
    return out if n_out == 2 else (out[0], None)


def kernel(x, edge_index, W0_1, b0_1, W0_2, b0_2, W0_3, b0_3,
           W1_1, b1_1, W1_2, b1_2, W1_3, b1_3, W_head, b_head):
    src = edge_index[0].reshape(NW, NGRP, G, K)
    dst = edge_index[1].reshape(NW, NGRP, G, K)
    z128 = jnp.zeros((RPT, D), jnp.float32)
    ones128 = jnp.ones((K, D), jnp.float32)

    dg = _sc_degree(dst, ones128, z128)
    dinv, g0 = _deg_finish(dg[0], dg[1], x)

    def prop(g):
        p = _sc_propagate(g, src, dst, z128)
        return p[0], p[1]

    pa, pb = prop(g0)
    h1, g1 = _combine(pa, pb, g0, dinv)
    pa, pb = prop(g1)
    h2, g2 = _combine(pa, pb, g1, dinv)
    pa, pb = prop(g2)
    h3, _ = _combine(pa, pb, g2, dinv)

    _, gh1 = _mm(h1, W0_1, b0_1, dinv=dinv, out_relu=True)
    _, gh2 = _mm(h2, W0_2, b0_2, dinv=dinv, out_relu=True)
    _, gh3 = _mm(h3, W0_3, b0_3, dinv=dinv, out_relu=True)

    t1_blocks = []
    for gh in (gh1, gh2, gh3):
        pa, pb = prop(gh)
        tb, _ = _combine(pa, pb, gh, dinv)
        t1_blocks.append(tb)
    t1 = jnp.concatenate(t1_blocks, axis=1)

    out1, _ = _mm(t1, W1_1)
    _, gu1 = _mm(t1, W1_2, dinv=dinv)
    _, gu2 = _mm(t1, W1_3, dinv=dinv)

    pa, pb = prop(gu1)
    out2, _ = _combine(pa, pb, gu1, dinv)
    pa, pb = prop(gu2)
    _, gv = _combine(pa, pb, gu2, dinv)
    pa, pb = prop(gv)
    out3, _ = _combine(pa, pb, gv, dinv)

    h2pre = jnp.concatenate([out1, out2, out3], axis=1)
    bcat = jnp.concatenate([b1_1, b1_2, b1_3])
    y, _ = _mm(h2pre, W_head, b_head, b_in=bcat, in_relu=True)
    return y
